# baseline (speedup 1.0000x reference)
import functools

import jax
import jax.numpy as jnp
from jax import lax
from jax.experimental import pallas as pl
from jax.experimental.pallas import tpu as pltpu
from jax.experimental.pallas import tpu_sc as plsc

_N = 8
_E = 1048576
_NS = 16
_L = 16
_EPT = _E // _NS
_NP = 8
_P = _EPT // _NP
_UNROLL = 8
_NBINS = 64

_OFF_X = 0
_OFF_W1 = 8192
_OFF_B1 = 10240
_OFF_W2 = 10256
_OFF_B2 = 10288
_OFF_W3 = 10304
_OFF_B3 = 10432
_OFF_WL1 = 10448
_OFF_BL1 = 18640
_OFF_WL2 = 18704
_OFF_BL2 = 19728
_PSIZE = 19744


def _pack_params(x, W_c1, b_c1, W_c2, b_c2, W_c3, b_c3, W_l1, b_l1,
                 W_l2, b_l2):
    xp = jnp.zeros((16, 512), jnp.float32).at[:8, :].set(x).T

    def pad16(v):
        v = v.reshape(-1)
        n = (-v.size) % 16
        return jnp.pad(v, (0, n))

    return jnp.concatenate([
        xp.reshape(-1), W_c1.reshape(-1), pad16(b_c1), pad16(W_c2),
        pad16(b_c2), W_c3.reshape(-1), b_c3, W_l1.T.reshape(-1), b_l1,
        W_l2.T.reshape(-1), b_l2,
    ])


def _newton_rsqrt(v):
    i = plsc.bitcast(v, jnp.int32)
    y = plsc.bitcast(jnp.int32(0x5F3759DF) - (i >> 1), jnp.float32)
    for _ in range(4):
        y = y * (1.5 - 0.5 * v * y * y)
    return y


def _sc_gnn(edge_index, params):
    mesh = plsc.VectorSubcoreMesh(core_axis_name="c", subcore_axis_name="s")

    @functools.partial(
        pl.kernel,
        out_type=jax.ShapeDtypeStruct((_L,), jnp.float32),
        mesh=mesh,
        scratch_types=[
            pltpu.VMEM((2, _P), jnp.int32),
            pltpu.VMEM((2, _P), jnp.int32),
            pltpu.VMEM((_L * _NBINS,), jnp.float32),
            pltpu.VMEM((_NBINS,), jnp.float32),
            pltpu.VMEM((_PSIZE,), jnp.float32),
            pltpu.VMEM((_NS * _NBINS,), jnp.float32),
            pltpu.VMEM((96,), jnp.float32),
            pltpu.VMEM((16,), jnp.float32),
            pltpu.VMEM((16 * 16,), jnp.float32),
            pltpu.VMEM((16 * 16,), jnp.float32),
            pltpu.VMEM((64,), jnp.float32),
            pltpu.VMEM_SHARED((_NS * _NBINS,), jnp.float32),
            pltpu.SemaphoreType.DMA,
            pltpu.SemaphoreType.DMA,
            pltpu.SemaphoreType.DMA,
            pltpu.SemaphoreType.DMA,
            pltpu.SemaphoreType.DMA,
        ],
        compiler_params=pltpu.CompilerParams(needs_layout_passes=False),
    )
    def gnn(edge_hbm, par_hbm, out_hbm, rows_v, cols_v, acc_v, cnt_v,
            par_v, all_v, cbuf, dbuf, hA, hB, qbuf, shared,
            sr0, sr1, sc0, sc1, sp):
        cid = lax.axis_index("c")
        sid = lax.axis_index("s")

        @pl.when(cid == 0)
        def _():
            base = sid * _EPT
            sems_r = (sr0, sr1)
            sems_c = (sc0, sc1)

            @pl.when(sid == 0)
            def _():
                pltpu.async_copy(par_hbm, par_v, sp)

            def start(p):
                slot = p % 2
                off = base + p * _P
                return (
                    pltpu.async_copy(edge_hbm.at[0, pl.ds(off, _P)],
                                     rows_v.at[slot], sems_r[slot]),
                    pltpu.async_copy(edge_hbm.at[1, pl.ds(off, _P)],
                                     cols_v.at[slot], sems_c[slot]),
                )

            inflight = start(0)

            zeros = jnp.zeros((_L,), jnp.float32)
            for j in range(_NBINS):
                acc_v[pl.ds(j * _L, _L)] = zeros

            iot = lax.iota(jnp.int32, _L)
            lane_base = iot * _NBINS
            ones = jnp.ones((_L,), jnp.float32)

            for p in range(_NP):
                cur = inflight
                if p + 1 < _NP:
                    inflight = start(p + 1)
                cur[0].wait()
                cur[1].wait()
                slot = p % 2

                def body(i, carry):
                    base_i = i * (_L * _UNROLL)
                    for u in range(_UNROLL):
                        r = rows_v[slot, pl.ds(base_i + u * _L, _L)]
                        c = cols_v[slot, pl.ds(base_i + u * _L, _L)]
                        addr = lane_base + r * _N + c
                        plsc.addupdate_scatter(acc_v, [addr], ones)
                    return carry

                lax.fori_loop(0, _P // (_L * _UNROLL), body, 0)

            for kk in range(_NBINS // _L):
                s = acc_v[pl.ds(kk * _L, _L)]
                for l in range(1, _L):
                    s = s + acc_v[pl.ds(l * _NBINS + kk * _L, _L)]
                cnt_v[pl.ds(kk * _L, _L)] = s

            pltpu.sync_copy(cnt_v, shared.at[pl.ds(sid * _NBINS, _NBINS)])
            plsc.subcore_barrier()

            @pl.when(sid == 0)
            def _():
                pltpu.sync_copy(shared, all_v)
                pltpu.make_async_copy(par_hbm, par_v, sp).wait()
                iotf = iot.astype(jnp.float32)

                for kk in range(_NBINS // _L):
                    s = all_v[pl.ds(kk * _L, _L)]
                    for t in range(1, _NS):
                        s = s + all_v[pl.ds(t * _NBINS + kk * _L, _L)]
                    cbuf[pl.ds(kk * _L, _L)] = s
                cbuf[pl.ds(64, _L)] = zeros
                cbuf[pl.ds(80, _L)] = zeros

                def vgat(v, idx):
                    dn = lax.GatherDimensionNumbers(
                        offset_dims=(), collapsed_slice_dims=(0,),
                        start_index_map=(0,))
                    return lax.gather(
                        v, idx[:, None], dn, (1,),
                        mode=lax.GatherScatterMode.PROMISE_IN_BOUNDS)

                def lane(i):
                    return jnp.full((_L,), i, jnp.int32)

                def spl(ref, i):
                    w = ref[pl.ds((i // _L) * _L, _L)]
                    return vgat(w, lane(i % _L))

                wins = [cbuf[pl.ds(q * _L, _L)] for q in range(4)]
                crows = []
                deg = ones
                for r in range(_N):
                    cr = vgat(wins[r // 2], (iot + 8 * (r % 2)) & 15)
                    crows.append(cr)
                    deg = deg + cr
                dis = _newton_rsqrt(deg)
                dbuf[...] = dis

                acols = []
                for r in range(_N):
                    oneh = (iot == r).astype(jnp.float32)
                    dr = spl(dbuf, r)
                    acols.append(dis * (crows[r] + oneh) * dr)

                def gbody(kc, carry):
                    gg = list(carry)
                    wch = [par_v[pl.ds(_OFF_W1 + j * 512 + kc * _L, _L)]
                           for j in range(4)]
                    for t in range(_L):
                        xc = par_v[pl.ds(_OFF_X + kc * 256 + t * _L, _L)]
                        for j in range(4):
                            gg[j] = gg[j] + xc * vgat(wch[j], lane(t))
                    return tuple(gg)

                g = list(lax.fori_loop(
                    0, 32, gbody,
                    tuple(jnp.zeros((_L,), jnp.float32) for _ in range(4))))

                def conv_relu(feats, off_b, hbuf):
                    for j, fv in enumerate(feats):
                        hbuf[pl.ds(j * _L, _L)] = fv
                    out = []
                    for j in range(len(feats)):
                        accv = spl(par_v, off_b + j)
                        for r in range(_N):
                            fr = spl(hbuf, j * _L + r)
                            accv = accv + acols[r] * fr
                        out.append(jnp.maximum(accv, 0.0))
                    return out

                def linmix(feats, off_w, f_out):
                    f_in = len(feats)
                    out = []
                    for j in range(f_out):
                        accv = jnp.zeros((_L,), jnp.float32)
                        for f in range(f_in):
                            wv = spl(par_v, off_w + j * f_in + f)
                            accv = accv + feats[f] * wv
                        out.append(accv)
                    return out

                h1 = conv_relu(g, _OFF_B1, hA)
                u2 = linmix(h1, _OFF_W2, 8)
                h2 = conv_relu(u2, _OFF_B2, hA)
                u3 = linmix(h2, _OFF_W3, 16)
                h3 = conv_relu(u3, _OFF_B3, hA)

                for f, fv in enumerate(h3):
                    hB[pl.ds(f * _L, _L)] = fv

                lin = [par_v[pl.ds(_OFF_BL1 + q * _L, _L)] for q in range(4)]
                for n in range(_N):
                    for f in range(16):
                        hv = spl(hB, f * _L + n)
                        rowoff = _OFF_WL1 + (n * 16 + f) * 64
                        for q in range(4):
                            wv = par_v[pl.ds(rowoff + q * _L, _L)]
                            lin[q] = lin[q] + hv * wv
                for q in range(4):
                    qbuf[pl.ds(q * _L, _L)] = jnp.maximum(lin[q], 0.0)

                out = par_v[pl.ds(_OFF_BL2, _L)]
                for j in range(64):
                    qv = spl(qbuf, j)
                    wv = par_v[pl.ds(_OFF_WL2 + j * _L, _L)]
                    out = out + qv * wv

                dbuf[...] = out
                pltpu.sync_copy(dbuf, out_hbm)

    return gnn(edge_index, params)


def kernel(x, edge_index, W_c1, b_c1, W_c2, b_c2, W_c3, b_c3,
           W_l1, b_l1, W_l2, b_l2):
    params = _pack_params(x, W_c1, b_c1, W_c2, b_c2, W_c3, b_c3,
                          W_l1, b_l1, W_l2, b_l2)
    out = _sc_gnn(edge_index, params)
    return out.reshape(1, 16)


# speedup vs baseline: 1.5873x; 1.5873x over previous
import functools

import jax
import jax.numpy as jnp
from jax import lax
from jax.experimental import pallas as pl
from jax.experimental.pallas import tpu as pltpu
from jax.experimental.pallas import tpu_sc as plsc

_N = 8
_E = 1048576
_NC = 2
_NS = 16
_NW = _NC * _NS
_L = 16
_CH = _E // _NW
_NBINS = _N * _N
_UNROLL = 8
_STRIDE = 65
_NP = 4
_P = _CH // _NP


def _sc_edge_histogram(edge_index_2d):
    mesh = plsc.VectorSubcoreMesh(core_axis_name="c", subcore_axis_name="s")

    @functools.partial(
        pl.kernel,
        out_type=jax.ShapeDtypeStruct((_NW * _NBINS,), jnp.float32),
        mesh=mesh,
        scratch_types=[
            pltpu.VMEM((2, _P), jnp.int32),
            pltpu.VMEM((2, _P), jnp.int32),
            pltpu.VMEM((2 * _L * _STRIDE,), jnp.float32),
            pltpu.VMEM((_NBINS,), jnp.float32),
            pltpu.SemaphoreType.DMA,
            pltpu.SemaphoreType.DMA,
            pltpu.SemaphoreType.DMA,
            pltpu.SemaphoreType.DMA,
        ],
        compiler_params=pltpu.CompilerParams(needs_layout_passes=False),
    )
    def hist(edge_hbm, out_hbm, rows_v, cols_v, acc_v, cnt_v,
             sr0, sr1, sc0, sc1):
        wid = lax.axis_index("s") * _NC + lax.axis_index("c")
        base = wid * _CH
        sems_r = (sr0, sr1)
        sems_c = (sc0, sc1)

        def start(p):
            slot = p % 2
            off = base + p * _P
            return (
                pltpu.async_copy(edge_hbm.at[0, pl.ds(off, _P)],
                                 rows_v.at[slot], sems_r[slot]),
                pltpu.async_copy(edge_hbm.at[1, pl.ds(off, _P)],
                                 cols_v.at[slot], sems_c[slot]),
            )

        inflight = start(0)

        zeros = jnp.zeros((_L,), jnp.float32)
        for j in range(2 * _L * _STRIDE // _L):
            acc_v[pl.ds(j * _L, _L)] = zeros

        lane_bases = (lax.iota(jnp.int32, _L) * _STRIDE,
                      lax.iota(jnp.int32, _L) * _STRIDE + _L * _STRIDE)
        ones = jnp.ones((_L,), jnp.float32)

        for p in range(_NP):
            cur = inflight
            if p + 1 < _NP:
                inflight = start(p + 1)
            cur[0].wait()
            cur[1].wait()
            slot = p % 2

            def body(i, carry):
                base_i = i * (_L * _UNROLL)
                for u in range(_UNROLL):
                    r = rows_v[slot, pl.ds(base_i + u * _L, _L)]
                    c = cols_v[slot, pl.ds(base_i + u * _L, _L)]
                    addr = lane_bases[u % 2] + r * _N + c
                    plsc.addupdate_scatter(acc_v, [addr], ones)
                return carry

            lax.fori_loop(0, _P // (_L * _UNROLL), body, 0)

        for kk in range(_NBINS // _L):
            s = acc_v[pl.ds(kk * _L, _L)]
            for a in range(2):
                for l in range(_L):
                    if a == 0 and l == 0:
                        continue
                    off = a * _L * _STRIDE + l * _STRIDE + kk * _L
                    s = s + acc_v[pl.ds(off, _L)]
            cnt_v[pl.ds(kk * _L, _L)] = s

        pltpu.sync_copy(cnt_v, out_hbm.at[pl.ds(wid * _NBINS, _NBINS)])

    return hist(edge_index_2d)


def _tc_head(parts, x, W1, b1, W2, b2, W3, b3, Wl1t, bl1, Wl2, bl2):
    def body(p_ref, x_ref, w1_ref, b1_ref, w2_ref, b2_ref, w3_ref, b3_ref,
             wl1_ref, bl1_ref, wl2_ref, bl2_ref, o_ref):
        C = jnp.sum(p_ref[...], axis=0)
        ii = lax.broadcasted_iota(jnp.int32, (_N, _N), 0)
        jj = lax.broadcasted_iota(jnp.int32, (_N, _N), 1)
        C = C + (ii == jj).astype(jnp.float32)
        deg = jnp.sum(C, axis=0, keepdims=True)
        dis = lax.rsqrt(deg)
        Dm = (ii == jj).astype(jnp.float32) * dis

        def dot(a, b, dims):
            return lax.dot_general(a, b, (dims, ((), ())),
                                   preferred_element_type=jnp.float32)

        A = dot(Dm, dot(C, Dm, (((0,), (0,)))), (((1,), (0,))))

        h = dot(x_ref[...], w1_ref[...], (((1,), (1,))))
        h = jnp.maximum(dot(A, h, (((1,), (0,)))) + b1_ref[...], 0.0)
        h = dot(h, w2_ref[...], (((1,), (1,))))
        h = jnp.maximum(dot(A, h, (((1,), (0,)))) + b2_ref[...], 0.0)
        h = dot(h, w3_ref[...], (((1,), (1,))))
        h = jnp.maximum(dot(A, h, (((1,), (0,)))) + b3_ref[...], 0.0)

        lin1 = bl1_ref[...]
        for n in range(_N):
            lin1 = lin1 + dot(h[n:n + 1, :],
                              wl1_ref[n * 16:(n + 1) * 16, :],
                              (((1,), (0,))))
        q = jnp.maximum(lin1, 0.0)
        o_ref[...] = dot(q, wl2_ref[...], (((1,), (1,)))) + bl2_ref[...]

    return pl.pallas_call(
        body,
        out_shape=jax.ShapeDtypeStruct((1, 16), jnp.float32),
    )(parts, x, W1, b1, W2, b2, W3, b3, Wl1t, bl1, Wl2, bl2)


def kernel(x, edge_index, W_c1, b_c1, W_c2, b_c2, W_c3, b_c3,
           W_l1, b_l1, W_l2, b_l2):
    parts = _sc_edge_histogram(edge_index)
    return _tc_head(
        parts.reshape(_NW, _N, _N), x,
        W_c1, b_c1.reshape(1, -1),
        W_c2, b_c2.reshape(1, -1),
        W_c3, b_c3.reshape(1, -1),
        W_l1.T, b_l1.reshape(1, -1),
        W_l2, b_l2.reshape(1, -1),
    )


# speedup vs baseline: 1.6028x; 1.0098x over previous
import functools

import jax
import jax.numpy as jnp
from jax import lax
from jax.experimental import pallas as pl
from jax.experimental.pallas import tpu as pltpu
from jax.experimental.pallas import tpu_sc as plsc

_N = 8
_E = 1048576
_NC = 2
_NS = 16
_NW = _NC * _NS
_L = 16
_CH = _E // _NW
_NBINS = _N * _N
_UNROLL = 8
_NP = 4
_P = _CH // _NP


def _sc_edge_histogram(edge_index_2d):
    mesh = plsc.VectorSubcoreMesh(core_axis_name="c", subcore_axis_name="s")

    @functools.partial(
        pl.kernel,
        out_type=jax.ShapeDtypeStruct((_NW * _NBINS,), jnp.float32),
        mesh=mesh,
        scratch_types=[
            pltpu.VMEM((2, _P), jnp.int32),
            pltpu.VMEM((2, _P), jnp.int32),
            pltpu.VMEM((_L * _NBINS,), jnp.float32),
            pltpu.VMEM((_NBINS,), jnp.float32),
            pltpu.SemaphoreType.DMA,
            pltpu.SemaphoreType.DMA,
            pltpu.SemaphoreType.DMA,
            pltpu.SemaphoreType.DMA,
        ],
        compiler_params=pltpu.CompilerParams(needs_layout_passes=False),
    )
    def hist(edge_hbm, out_hbm, rows_v, cols_v, acc_v, cnt_v,
             sr0, sr1, sc0, sc1):
        wid = lax.axis_index("s") * _NC + lax.axis_index("c")
        base = wid * _CH
        sems_r = (sr0, sr1)
        sems_c = (sc0, sc1)

        def start(p):
            slot = p % 2
            off = base + p * _P
            return (
                pltpu.async_copy(edge_hbm.at[0, pl.ds(off, _P)],
                                 rows_v.at[slot], sems_r[slot]),
                pltpu.async_copy(edge_hbm.at[1, pl.ds(off, _P)],
                                 cols_v.at[slot], sems_c[slot]),
            )

        inflight = start(0)

        zeros = jnp.zeros((_L,), jnp.float32)
        for j in range(_NBINS):
            acc_v[pl.ds(j * _L, _L)] = zeros

        lane_base = lax.iota(jnp.int32, _L) * _NBINS
        ones = jnp.ones((_L,), jnp.float32)

        for p in range(_NP):
            cur = inflight
            if p + 1 < _NP:
                inflight = start(p + 1)
            cur[0].wait()
            cur[1].wait()
            slot = p % 2

            def body(i, carry):
                base_i = i * (_L * _UNROLL)
                for u in range(_UNROLL):
                    r = rows_v[slot, pl.ds(base_i + u * _L, _L)]
                    c = cols_v[slot, pl.ds(base_i + u * _L, _L)]
                    addr = lane_base + r * _N + c
                    plsc.addupdate_scatter(acc_v, [addr], ones)
                return carry

            lax.fori_loop(0, _P // (_L * _UNROLL), body, 0)

        for kk in range(_NBINS // _L):
            s = acc_v[pl.ds(kk * _L, _L)]
            for l in range(1, _L):
                s = s + acc_v[pl.ds(l * _NBINS + kk * _L, _L)]
            cnt_v[pl.ds(kk * _L, _L)] = s

        pltpu.sync_copy(cnt_v, out_hbm.at[pl.ds(wid * _NBINS, _NBINS)])

    return hist(edge_index_2d)


def _tc_head(parts, x, W1, b1, W2, b2, W3, b3, Wl1t, bl1, Wl2, bl2):
    def body(p_ref, x_ref, w1_ref, b1_ref, w2_ref, b2_ref, w3_ref, b3_ref,
             wl1_ref, bl1_ref, wl2_ref, bl2_ref, o_ref):
        C = jnp.sum(p_ref[...], axis=0)
        ii = lax.broadcasted_iota(jnp.int32, (_N, _N), 0)
        jj = lax.broadcasted_iota(jnp.int32, (_N, _N), 1)
        C = C + (ii == jj).astype(jnp.float32)
        deg = jnp.sum(C, axis=0, keepdims=True)
        dis = lax.rsqrt(deg)
        Dm = (ii == jj).astype(jnp.float32) * dis

        def dot(a, b, dims):
            return lax.dot_general(a, b, (dims, ((), ())),
                                   preferred_element_type=jnp.float32)

        A = dot(Dm, dot(C, Dm, (((0,), (0,)))), (((1,), (0,))))

        h = dot(x_ref[...], w1_ref[...], (((1,), (1,))))
        h = jnp.maximum(dot(A, h, (((1,), (0,)))) + b1_ref[...], 0.0)
        h = dot(h, w2_ref[...], (((1,), (1,))))
        h = jnp.maximum(dot(A, h, (((1,), (0,)))) + b2_ref[...], 0.0)
        h = dot(h, w3_ref[...], (((1,), (1,))))
        h = jnp.maximum(dot(A, h, (((1,), (0,)))) + b3_ref[...], 0.0)

        lin1 = bl1_ref[...]
        for n in range(_N):
            lin1 = lin1 + dot(h[n:n + 1, :],
                              wl1_ref[n * 16:(n + 1) * 16, :],
                              (((1,), (0,))))
        q = jnp.maximum(lin1, 0.0)
        o_ref[...] = dot(q, wl2_ref[...], (((1,), (1,)))) + bl2_ref[...]

    return pl.pallas_call(
        body,
        out_shape=jax.ShapeDtypeStruct((1, 16), jnp.float32),
    )(parts, x, W1, b1, W2, b2, W3, b3, Wl1t, bl1, Wl2, bl2)


def kernel(x, edge_index, W_c1, b_c1, W_c2, b_c2, W_c3, b_c3,
           W_l1, b_l1, W_l2, b_l2):
    parts = _sc_edge_histogram(edge_index)
    return _tc_head(
        parts.reshape(_NW, _N, _N), x,
        W_c1, b_c1.reshape(1, -1),
        W_c2, b_c2.reshape(1, -1),
        W_c3, b_c3.reshape(1, -1),
        W_l1.T, b_l1.reshape(1, -1),
        W_l2, b_l2.reshape(1, -1),
    )
